# Initial kernel scaffold; baseline (speedup 1.0000x reference)
import jax
import jax.numpy as jnp
from jax.experimental import pallas as pl


def kernel(x, batch, W, b, pe_table):
    raise NotImplementedError("write your pallas kernel here")


# speedup vs baseline: 2.6217x; 2.6217x over previous
import functools

import jax
import jax.numpy as jnp
from jax.experimental import pallas as pl

_N = 100000
_BLOCK = 2000
_PARTS_PAD = 32


def _fused_kernel(x_ref, ids_ref, w_ref, t_ref, b_ref, o_ref):
    ids = ids_ref[...]
    one_hot = (
        ids == jax.lax.broadcasted_iota(jnp.int32, (ids.shape[0], _PARTS_PAD), 1)
    ).astype(jnp.float32)
    acc = jnp.dot(x_ref[...], w_ref[...], preferred_element_type=jnp.float32)
    acc = acc + jnp.dot(one_hot, t_ref[...], preferred_element_type=jnp.float32)
    o_ref[...] = acc + b_ref[...]


@jax.jit
def kernel(x, batch, W, b, pe_table):
    n, d_in = x.shape
    d_out = W.shape[1]
    d_pe = pe_table.shape[1]
    d_emb = d_out + d_pe

    w_pad = jnp.zeros((d_in, d_emb), jnp.float32).at[:, :d_out].set(W)
    t_pad = (
        jnp.zeros((_PARTS_PAD, d_emb), jnp.float32)
        .at[: pe_table.shape[0], d_out:]
        .set(pe_table)
    )
    b_pad = jnp.zeros((1, d_emb), jnp.float32).at[0, :d_out].set(b)
    ids2d = batch.astype(jnp.int32).reshape(n, 1)

    block = _BLOCK if n % _BLOCK == 0 else 8
    grid = n // block

    return pl.pallas_call(
        _fused_kernel,
        grid=(grid,),
        in_specs=[
            pl.BlockSpec((block, d_in), lambda i: (i, 0)),
            pl.BlockSpec((block, 1), lambda i: (i, 0)),
            pl.BlockSpec((d_in, d_emb), lambda i: (0, 0)),
            pl.BlockSpec((_PARTS_PAD, d_emb), lambda i: (0, 0)),
            pl.BlockSpec((1, d_emb), lambda i: (0, 0)),
        ],
        out_specs=pl.BlockSpec((block, d_emb), lambda i: (i, 0)),
        out_shape=jax.ShapeDtypeStruct((n, d_emb), jnp.float32),
    )(x, ids2d, w_pad, t_pad, b_pad)
